# Initial kernel scaffold; baseline (speedup 1.0000x reference)
import jax
import jax.numpy as jnp
from jax.experimental import pallas as pl


def kernel(z, embeddings):
    raise NotImplementedError("write your pallas kernel here")


# speedup vs baseline: 2.9427x; 2.9427x over previous
import jax
import jax.numpy as jnp
from jax.experimental import pallas as pl
from jax.experimental.pallas import tpu as pltpu

_NE = 1024
_D = 64
_CC = 0.25
_BM = 512


def _vq_body(z_ref, e_ref, enc_ref, q_ref, loss_ref, perp_ref,
             cnt_acc, loss_acc):
    i = pl.program_id(0)
    nsteps = pl.num_programs(0)
    n_tok = nsteps * _BM

    z = z_ref[...]
    e = e_ref[...]
    sz2 = jnp.sum(z * z, axis=1, keepdims=True)
    se2 = jnp.sum(e * e, axis=1)
    mm = jax.lax.dot_general(z, e, (((1,), (1,)), ((), ())),
                             preferred_element_type=jnp.float32)
    d = (sz2 + se2[None, :]) - 2.0 * mm

    dmin = jnp.min(d, axis=1, keepdims=True)
    col = jax.lax.broadcasted_iota(jnp.int32, (_BM, _NE), 1)
    idx = jnp.min(jnp.where(d == dmin, col, _NE), axis=1, keepdims=True)
    enc = (col == idx).astype(jnp.float32)
    enc_ref[...] = enc

    q = jax.lax.dot_general(enc, e, (((1,), (0,)), ((), ())),
                            preferred_element_type=jnp.float32)
    q_ref[...] = q

    diff = q - z
    tile_loss = jnp.sum(diff * diff)
    tile_cnt = jnp.sum(enc, axis=0, keepdims=True)

    @pl.when(i == 0)
    def _init():
        cnt_acc[...] = tile_cnt
        loss_acc[0, 0] = tile_loss

    @pl.when(i > 0)
    def _accum():
        cnt_acc[...] += tile_cnt
        loss_acc[0, 0] += tile_loss

    @pl.when(i == nsteps - 1)
    def _finalize():
        avg = cnt_acc[...] * (1.0 / n_tok)
        perp_ref[0, 0] = jnp.exp(-jnp.sum(avg * jnp.log(avg + 1e-10)))
        loss_ref[0, 0] = (1.0 + _CC) * loss_acc[0, 0] / (n_tok * _D)


def kernel(z, embeddings):
    b, c, h, w = z.shape
    z_flat = jnp.transpose(z, (0, 2, 3, 1)).reshape(-1, _D)
    n_tok = z_flat.shape[0]
    grid = (n_tok // _BM,)

    enc, q, loss, perp = pl.pallas_call(
        _vq_body,
        grid=grid,
        in_specs=[
            pl.BlockSpec((_BM, _D), lambda i: (i, 0)),
            pl.BlockSpec((_NE, _D), lambda i: (0, 0)),
        ],
        out_specs=[
            pl.BlockSpec((_BM, _NE), lambda i: (i, 0)),
            pl.BlockSpec((_BM, _D), lambda i: (i, 0)),
            pl.BlockSpec(memory_space=pltpu.SMEM),
            pl.BlockSpec(memory_space=pltpu.SMEM),
        ],
        out_shape=[
            jax.ShapeDtypeStruct((n_tok, _NE), jnp.float32),
            jax.ShapeDtypeStruct((n_tok, _D), jnp.float32),
            jax.ShapeDtypeStruct((1, 1), jnp.float32),
            jax.ShapeDtypeStruct((1, 1), jnp.float32),
        ],
        scratch_shapes=[
            pltpu.VMEM((1, _NE), jnp.float32),
            pltpu.SMEM((1, 1), jnp.float32),
        ],
    )(z_flat, embeddings)

    quantized = q.reshape(b, c, h, w)
    return (quantized, loss[0, 0], perp[0, 0], enc)
